# baseline (speedup 1.0000x reference)
import jax
import jax.numpy as jnp
from jax import lax
from jax.experimental import pallas as pl
from jax.experimental.pallas import tpu as pltpu
from jax.experimental.pallas import tpu_sc as plsc

VOCAB = 100000
EMBED_DIM = 128
BATCH = 1024
CTX = 20
MAX_NORM = 1.0

_NC = 2
_NS = 16
_NW = _NC * _NS
_LANES = 16
_VPR = EMBED_DIM // _LANES

_ROWS = BATCH * CTX
_ROWS_W = _ROWS // _NW
_B_W = BATCH // _NW
_CHUNK = 128
_NCHUNK = _ROWS_W // _CHUNK


def _rsqrt_vec(ss):
    i = lax.bitcast_convert_type(ss, jnp.int32)
    y = lax.bitcast_convert_type(
        jnp.full((_LANES,), 0x5F3759DF, jnp.int32)
        - lax.shift_right_arithmetic(i, jnp.full((_LANES,), 1, jnp.int32)),
        jnp.float32)
    for _ in range(3):
        y = y * (jnp.float32(1.5) - jnp.float32(0.5) * ss * y * y)
    return y


def _lane_gather(v, idx):
    return lax.gather(
        v, idx[:, None],
        lax.GatherDimensionNumbers(
            offset_dims=(), collapsed_slice_dims=(0,), start_index_map=(0,)),
        slice_sizes=(1,),
        mode=lax.GatherScatterMode.PROMISE_IN_BOUNDS)


def _tree_reduce_sum(v):
    for sh in (1, 2, 4, 8):
        idx = (lax.iota(jnp.int32, _LANES) + sh) % _LANES
        v = v + _lane_gather(v, idx)
    return v


def _pool_body(idx_hbm, table_hbm, out_hbm, idx_v, rows_v, acc_v, sem):
    wid = lax.axis_index("s") * _NC + lax.axis_index("c")

    pltpu.sync_copy(idx_hbm.at[wid], idx_v)

    copies = [
        pltpu.async_copy(
            table_hbm.at[idx_v.at[j]],
            rows_v.at[pl.ds(j * _CHUNK, _CHUNK)],
            sem,
        )
        for j in range(_NCHUNK)
    ]
    for cp in copies:
        cp.wait()

    inv_ctx = jnp.float32(1.0 / CTX)

    def body(bi, _):
        base_row = bi * CTX
        accs = [jnp.zeros((_LANES,), jnp.float32) for _ in range(_VPR)]
        for j in range(CTX):
            row = rows_v.at[base_row + j]
            vs = [row[pl.ds(k * _LANES, _LANES)] for k in range(_VPR)]
            ssv = vs[0] * vs[0]
            for k in range(1, _VPR):
                ssv = ssv + vs[k] * vs[k]
            ss = _tree_reduce_sum(ssv)
            norm = ss * _rsqrt_vec(ss)
            scale = jnp.minimum(jnp.full((_LANES,), MAX_NORM, jnp.float32),
                                jnp.float32(MAX_NORM) / (norm + jnp.float32(1e-7)))
            accs = [a + v * scale for a, v in zip(accs, vs)]
        for k in range(_VPR):
            acc_v[bi, pl.ds(k * _LANES, _LANES)] = accs[k] * inv_ctx
        return 0

    lax.fori_loop(0, _B_W, body, 0)

    pltpu.sync_copy(acc_v, out_hbm.at[pl.ds(wid * _B_W, _B_W)])


def _pool(idx, table):
    mesh = plsc.VectorSubcoreMesh(core_axis_name="c", subcore_axis_name="s")
    return pl.kernel(
        _pool_body,
        mesh=mesh,
        out_type=jax.ShapeDtypeStruct((BATCH, EMBED_DIM), jnp.float32),
        scratch_types=[
            pltpu.VMEM((_NCHUNK, _CHUNK), jnp.int32),
            pltpu.VMEM((_ROWS_W, EMBED_DIM), jnp.float32),
            pltpu.VMEM((_B_W, EMBED_DIM), jnp.float32),
            pltpu.SemaphoreType.DMA,
        ],
    )(idx, table)


_MB = 8


def _mm_body(x_ref, w_ref, b_ref, o_ref):
    o_ref[...] = lax.dot_general(
        x_ref[...], w_ref[...],
        (((1,), (1,)), ((), ())),
        preferred_element_type=jnp.float32,
    ) + b_ref[...]


def _logits(x, W, b2d):
    return pl.pallas_call(
        _mm_body,
        grid=(BATCH // _MB,),
        in_specs=[
            pl.BlockSpec((_MB, EMBED_DIM), lambda i: (i, 0)),
            pl.BlockSpec((VOCAB, EMBED_DIM), lambda i: (0, 0)),
            pl.BlockSpec((1, VOCAB), lambda i: (0, 0)),
        ],
        out_specs=pl.BlockSpec((_MB, VOCAB), lambda i: (i, 0)),
        out_shape=jax.ShapeDtypeStruct((BATCH, VOCAB), jnp.float32),
        compiler_params=pltpu.CompilerParams(
            dimension_semantics=("arbitrary",),
            vmem_limit_bytes=100 * 1024 * 1024,
        ),
    )(x, W, b2d)


def _probe_body(x_ref, o_ref):
    o_ref[...] = jnp.full(o_ref.shape, x_ref[0, 0], jnp.float32)


def _probe_strided(x):
    return pl.pallas_call(
        _probe_body,
        grid=(pl.cdiv(VOCAB, 2048),),
        in_specs=[pl.BlockSpec((8, EMBED_DIM), lambda i: (0, 0))],
        out_specs=pl.BlockSpec((BATCH, 2048), lambda i: (0, i)),
        out_shape=jax.ShapeDtypeStruct((BATCH, VOCAB), jnp.float32),
        compiler_params=pltpu.CompilerParams(
            dimension_semantics=("arbitrary",),
            vmem_limit_bytes=100 * 1024 * 1024,
        ),
    )(x)


def _probe_contig(x):
    return pl.pallas_call(
        _probe_body,
        grid=(BATCH // 64,),
        in_specs=[pl.BlockSpec((8, EMBED_DIM), lambda i: (0, 0))],
        out_specs=pl.BlockSpec((64, VOCAB), lambda i: (i, 0)),
        out_shape=jax.ShapeDtypeStruct((BATCH, VOCAB), jnp.float32),
        compiler_params=pltpu.CompilerParams(
            dimension_semantics=("arbitrary",),
            vmem_limit_bytes=100 * 1024 * 1024,
        ),
    )(x)


def kernel(input, table, W, b):
    idx = input.reshape(_NW, _NCHUNK, _CHUNK).astype(jnp.int32)
    x = _pool(idx, table)
    return _probe_contig(x)


# speedup vs baseline: 1.0054x; 1.0054x over previous
import jax
import jax.numpy as jnp
from jax import lax
from jax.experimental import pallas as pl
from jax.experimental.pallas import tpu as pltpu
from jax.experimental.pallas import tpu_sc as plsc

VOCAB = 100000
EMBED_DIM = 128
BATCH = 1024
CTX = 20
MAX_NORM = 1.0

_NC = 2
_NS = 16
_NW = _NC * _NS
_LANES = 16
_VPR = EMBED_DIM // _LANES

_ROWS = BATCH * CTX
_ROWS_W = _ROWS // _NW
_B_W = BATCH // _NW
_CHUNK = 128
_NCHUNK = _ROWS_W // _CHUNK


def _rsqrt_vec(ss):
    i = lax.bitcast_convert_type(ss, jnp.int32)
    y = lax.bitcast_convert_type(
        jnp.full((_LANES,), 0x5F3759DF, jnp.int32)
        - lax.shift_right_arithmetic(i, jnp.full((_LANES,), 1, jnp.int32)),
        jnp.float32)
    for _ in range(3):
        y = y * (jnp.float32(1.5) - jnp.float32(0.5) * ss * y * y)
    return y


def _lane_gather(v, idx):
    return lax.gather(
        v, idx[:, None],
        lax.GatherDimensionNumbers(
            offset_dims=(), collapsed_slice_dims=(0,), start_index_map=(0,)),
        slice_sizes=(1,),
        mode=lax.GatherScatterMode.PROMISE_IN_BOUNDS)


def _tree_reduce_sum(v):
    for sh in (1, 2, 4, 8):
        idx = (lax.iota(jnp.int32, _LANES) + sh) % _LANES
        v = v + _lane_gather(v, idx)
    return v


def _pool_body(idx_hbm, table_hbm, out_hbm, idx_v, rows_v, acc_v, sem):
    wid = lax.axis_index("s") * _NC + lax.axis_index("c")

    pltpu.sync_copy(idx_hbm.at[wid], idx_v)

    copies = [
        pltpu.async_copy(
            table_hbm.at[idx_v.at[j]],
            rows_v.at[pl.ds(j * _CHUNK, _CHUNK)],
            sem,
        )
        for j in range(_NCHUNK)
    ]
    for cp in copies:
        cp.wait()

    inv_ctx = jnp.float32(1.0 / CTX)

    def body(bi, _):
        base_row = bi * CTX
        accs = [jnp.zeros((_LANES,), jnp.float32) for _ in range(_VPR)]
        for j in range(CTX):
            row = rows_v.at[base_row + j]
            vs = [row[pl.ds(k * _LANES, _LANES)] for k in range(_VPR)]
            ssv = vs[0] * vs[0]
            for k in range(1, _VPR):
                ssv = ssv + vs[k] * vs[k]
            ss = _tree_reduce_sum(ssv)
            norm = ss * _rsqrt_vec(ss)
            scale = jnp.minimum(jnp.full((_LANES,), MAX_NORM, jnp.float32),
                                jnp.float32(MAX_NORM) / (norm + jnp.float32(1e-7)))
            accs = [a + v * scale for a, v in zip(accs, vs)]
        for k in range(_VPR):
            acc_v[bi, pl.ds(k * _LANES, _LANES)] = accs[k] * inv_ctx
        return 0

    lax.fori_loop(0, _B_W, body, 0)

    pltpu.sync_copy(acc_v, out_hbm.at[pl.ds(wid * _B_W, _B_W)])


def _pool(idx, table):
    mesh = plsc.VectorSubcoreMesh(core_axis_name="c", subcore_axis_name="s")
    return pl.kernel(
        _pool_body,
        mesh=mesh,
        out_type=jax.ShapeDtypeStruct((BATCH, EMBED_DIM), jnp.float32),
        scratch_types=[
            pltpu.VMEM((_NCHUNK, _CHUNK), jnp.int32),
            pltpu.VMEM((_ROWS_W, EMBED_DIM), jnp.float32),
            pltpu.VMEM((_B_W, EMBED_DIM), jnp.float32),
            pltpu.SemaphoreType.DMA,
        ],
    )(idx, table)


_MB = 8


def _mm_body(x_ref, w_ref, b_ref, o_ref):
    o_ref[...] = lax.dot_general(
        x_ref[...], w_ref[...],
        (((1,), (1,)), ((), ())),
        preferred_element_type=jnp.float32,
    ) + b_ref[...]


def _logits(x, W, b2d):
    return pl.pallas_call(
        _mm_body,
        grid=(BATCH // _MB,),
        in_specs=[
            pl.BlockSpec((_MB, EMBED_DIM), lambda i: (i, 0)),
            pl.BlockSpec((VOCAB, EMBED_DIM), lambda i: (0, 0)),
            pl.BlockSpec((1, VOCAB), lambda i: (0, 0)),
        ],
        out_specs=pl.BlockSpec((_MB, VOCAB), lambda i: (i, 0)),
        out_shape=jax.ShapeDtypeStruct((BATCH, VOCAB), jnp.float32),
        compiler_params=pltpu.CompilerParams(
            dimension_semantics=("arbitrary",),
            vmem_limit_bytes=100 * 1024 * 1024,
        ),
    )(x, W, b2d)


def _probe_body(x_ref, o_ref):
    o_ref[...] = jnp.full(o_ref.shape, x_ref[0, 0], jnp.float32)


def _probe_strided(x):
    return pl.pallas_call(
        _probe_body,
        grid=(pl.cdiv(VOCAB, 2048),),
        in_specs=[pl.BlockSpec((8, EMBED_DIM), lambda i: (0, 0))],
        out_specs=pl.BlockSpec((BATCH, 2048), lambda i: (0, i)),
        out_shape=jax.ShapeDtypeStruct((BATCH, VOCAB), jnp.float32),
        compiler_params=pltpu.CompilerParams(
            dimension_semantics=("arbitrary",),
            vmem_limit_bytes=100 * 1024 * 1024,
        ),
    )(x)


def _probe_contig(x):
    return pl.pallas_call(
        _probe_body,
        grid=(BATCH // 64,),
        in_specs=[pl.BlockSpec((8, EMBED_DIM), lambda i: (0, 0))],
        out_specs=pl.BlockSpec((64, VOCAB), lambda i: (i, 0)),
        out_shape=jax.ShapeDtypeStruct((BATCH, VOCAB), jnp.float32),
        compiler_params=pltpu.CompilerParams(
            dimension_semantics=("arbitrary",),
            vmem_limit_bytes=100 * 1024 * 1024,
        ),
    )(x)


def _probe_manual_body(x_ref, o_hbm, acc, sems):
    i = pl.program_id(0)
    slot = lax.rem(i, 2)

    @pl.when(i >= 2)
    def _():
        for q in range(4):
            pltpu.make_async_copy(
                acc.at[slot, pl.ds(q * 256, 256)],
                o_hbm.at[pl.ds(q * 256, 256), pl.ds(0, 2048)],
                sems.at[slot, q],
            ).wait()

    acc[slot] = jnp.full((BATCH, 2048), x_ref[0, 0], jnp.float32)
    col = i * 2048
    for q in range(4):
        pltpu.make_async_copy(
            acc.at[slot, pl.ds(q * 256, 256)],
            o_hbm.at[pl.ds(q * 256, 256), pl.ds(col, 2048)],
            sems.at[slot, q],
        ).start()

    @pl.when(i == 47)
    def _():
        for s in range(2):
            for q in range(4):
                pltpu.make_async_copy(
                    acc.at[s, pl.ds(q * 256, 256)],
                    o_hbm.at[pl.ds(q * 256, 256), pl.ds(0, 2048)],
                    sems.at[s, q],
                ).wait()


def _probe_manual(x):
    return pl.pallas_call(
        _probe_manual_body,
        grid=(48,),
        in_specs=[pl.BlockSpec((8, EMBED_DIM), lambda i: (0, 0))],
        out_specs=pl.BlockSpec(memory_space=pl.ANY),
        out_shape=jax.ShapeDtypeStruct((BATCH, VOCAB), jnp.float32),
        scratch_shapes=[
            pltpu.VMEM((2, BATCH, 2048), jnp.float32),
            pltpu.SemaphoreType.DMA((2, 4)),
        ],
        compiler_params=pltpu.CompilerParams(
            dimension_semantics=("arbitrary",),
            vmem_limit_bytes=100 * 1024 * 1024,
        ),
    )(x)


def kernel(input, table, W, b):
    idx = input.reshape(_NW, _NCHUNK, _CHUNK).astype(jnp.int32)
    x = _pool(idx, table)
    return _probe_manual(x)


# speedup vs baseline: 2.2934x; 2.2811x over previous
import jax
import jax.numpy as jnp
from jax import lax
from jax.experimental import pallas as pl
from jax.experimental.pallas import tpu as pltpu
from jax.experimental.pallas import tpu_sc as plsc

VOCAB = 100000
EMBED_DIM = 128
BATCH = 1024
CTX = 20
MAX_NORM = 1.0

_NC = 2
_NS = 16
_NW = _NC * _NS
_LANES = 16
_VPR = EMBED_DIM // _LANES

_ROWS = BATCH * CTX
_ROWS_W = _ROWS // _NW
_B_W = BATCH // _NW
_CHUNK = 128
_NCHUNK = _ROWS_W // _CHUNK


def _rsqrt_vec(ss):
    i = lax.bitcast_convert_type(ss, jnp.int32)
    y = lax.bitcast_convert_type(
        jnp.full((_LANES,), 0x5F3759DF, jnp.int32)
        - lax.shift_right_arithmetic(i, jnp.full((_LANES,), 1, jnp.int32)),
        jnp.float32)
    for _ in range(3):
        y = y * (jnp.float32(1.5) - jnp.float32(0.5) * ss * y * y)
    return y


def _lane_gather(v, idx):
    return lax.gather(
        v, idx[:, None],
        lax.GatherDimensionNumbers(
            offset_dims=(), collapsed_slice_dims=(0,), start_index_map=(0,)),
        slice_sizes=(1,),
        mode=lax.GatherScatterMode.PROMISE_IN_BOUNDS)


def _tree_reduce_sum(v):
    for sh in (1, 2, 4, 8):
        idx = (lax.iota(jnp.int32, _LANES) + sh) % _LANES
        v = v + _lane_gather(v, idx)
    return v


def _pool_body(idx_hbm, table_hbm, out_hbm, idx_v, rows_v, acc_v, sem):
    wid = lax.axis_index("s") * _NC + lax.axis_index("c")

    pltpu.sync_copy(idx_hbm.at[wid], idx_v)

    copies = [
        pltpu.async_copy(
            table_hbm.at[idx_v.at[j]],
            rows_v.at[pl.ds(j * _CHUNK, _CHUNK)],
            sem,
        )
        for j in range(_NCHUNK)
    ]
    for cp in copies:
        cp.wait()

    inv_ctx = jnp.float32(1.0 / CTX)

    def body(bi, _):
        base_row = bi * CTX
        accs = [jnp.zeros((_LANES,), jnp.float32) for _ in range(_VPR)]
        for j in range(CTX):
            row = rows_v.at[base_row + j]
            vs = [row[pl.ds(k * _LANES, _LANES)] for k in range(_VPR)]
            ssv = vs[0] * vs[0]
            for k in range(1, _VPR):
                ssv = ssv + vs[k] * vs[k]
            ss = _tree_reduce_sum(ssv)
            norm = ss * _rsqrt_vec(ss)
            scale = jnp.minimum(jnp.full((_LANES,), MAX_NORM, jnp.float32),
                                jnp.float32(MAX_NORM) / (norm + jnp.float32(1e-7)))
            accs = [a + v * scale for a, v in zip(accs, vs)]
        for k in range(_VPR):
            acc_v[bi, pl.ds(k * _LANES, _LANES)] = accs[k] * inv_ctx
        return 0

    lax.fori_loop(0, _B_W, body, 0)

    pltpu.sync_copy(acc_v, out_hbm.at[pl.ds(wid * _B_W, _B_W)])


def _pool(idx, table):
    mesh = plsc.VectorSubcoreMesh(core_axis_name="c", subcore_axis_name="s")
    return pl.kernel(
        _pool_body,
        mesh=mesh,
        out_type=jax.ShapeDtypeStruct((BATCH, EMBED_DIM), jnp.float32),
        scratch_types=[
            pltpu.VMEM((_NCHUNK, _CHUNK), jnp.int32),
            pltpu.VMEM((_ROWS_W, EMBED_DIM), jnp.float32),
            pltpu.VMEM((_B_W, EMBED_DIM), jnp.float32),
            pltpu.SemaphoreType.DMA,
        ],
    )(idx, table)


_VB = 2048


def _mm_body(w_ref, x_ref, b_ref, o_ref):
    o_ref[...] = lax.dot_general(
        w_ref[...], x_ref[...],
        (((1,), (1,)), ((), ())),
        preferred_element_type=jnp.float32,
    ) + b_ref[...]


def _logits_t(x, W, b2col):
    return pl.pallas_call(
        _mm_body,
        grid=(pl.cdiv(VOCAB, _VB),),
        in_specs=[
            pl.BlockSpec((_VB, EMBED_DIM), lambda i: (i, 0)),
            pl.BlockSpec((BATCH, EMBED_DIM), lambda i: (0, 0)),
            pl.BlockSpec((_VB, 1), lambda i: (i, 0)),
        ],
        out_specs=pl.BlockSpec((_VB, BATCH), lambda i: (i, 0)),
        out_shape=jax.ShapeDtypeStruct((VOCAB, BATCH), jnp.float32),
        compiler_params=pltpu.CompilerParams(
            dimension_semantics=("arbitrary",),
            vmem_limit_bytes=100 * 1024 * 1024,
        ),
    )(W, x, b2col)


def kernel(input, table, W, b):
    idx = input.reshape(_NW, _NCHUNK, _CHUNK).astype(jnp.int32)
    x = _pool(idx, table)
    return _logits_t(x, W, b.reshape(VOCAB, 1)).T
